# baseline (speedup 1.0000x reference)
import jax
import jax.numpy as jnp
from jax.experimental import pallas as pl
from jax.experimental.pallas import tpu as pltpu

_TIME_CHUNK = 16


def _lstm_seq_kernel(x_ref,
                     w_ref,
                     b_ref,
                     c0_ref,
                     h0_ref,
                     cy_ref,
                     hy_ref,
                     c_st, h_st):
    Tc = x_ref.shape[0]
    H = c0_ref.shape[1]

    @pl.when(pl.program_id(0) == 0)
    def _():
        c_st[...] = c0_ref[...]
        h_st[...] = h0_ref[...]

    def step(t, carry):
        c, h = carry
        xh = jnp.concatenate([x_ref[t].astype(jnp.bfloat16),
                              h.astype(jnp.bfloat16)], axis=1)
        gates = jnp.dot(xh, w_ref[...],
                        preferred_element_type=jnp.float32) + b_ref[...]
        ingate     = 0.5 * jnp.tanh(0.5 * gates[:, 0 * H:1 * H]) + 0.5
        forgetgate = 0.5 * jnp.tanh(0.5 * gates[:, 1 * H:2 * H]) + 0.5
        cellgate   = jnp.tanh(gates[:, 2 * H:3 * H])
        outgate    = 0.5 * jnp.tanh(0.5 * gates[:, 3 * H:4 * H]) + 0.5
        cy = forgetgate * c + ingate * cellgate
        hy = outgate * jnp.tanh(cy)
        cy_ref[t] = cy
        hy_ref[t] = hy
        return (cy, hy)

    cy, hy = jax.lax.fori_loop(0, Tc, step, (c_st[...], h_st[...]),
                               unroll=16)
    c_st[...] = cy
    h_st[...] = hy


def kernel(tokens, c0, h0, table, wi, bi, wh, bh):
    T, B = tokens.shape
    V, H = table.shape
    Tc = _TIME_CHUNK if T % _TIME_CHUNK == 0 else T

    x_emb = jnp.take(table, tokens, axis=0)
    b = bi + bh
    w16 = jnp.concatenate([wi, wh], axis=0).astype(jnp.bfloat16)

    cy_seq, hy_seq = pl.pallas_call(
        _lstm_seq_kernel,
        out_shape=(jax.ShapeDtypeStruct((T, B, H), jnp.float32),
                   jax.ShapeDtypeStruct((T, B, H), jnp.float32)),
        grid=(T // Tc,),
        in_specs=[
            pl.BlockSpec((Tc, B, H),  lambda i: (i, 0, 0)),
            pl.BlockSpec((2 * H, 4 * H), lambda i: (0, 0)),
            pl.BlockSpec((1, 4 * H),  lambda i: (0, 0)),
            pl.BlockSpec((B, H),      lambda i: (0, 0)),
            pl.BlockSpec((B, H),      lambda i: (0, 0)),
        ],
        out_specs=(pl.BlockSpec((Tc, B, H), lambda i: (i, 0, 0)),
                   pl.BlockSpec((Tc, B, H), lambda i: (i, 0, 0))),
        scratch_shapes=[
            pltpu.VMEM((B, H), jnp.float32),
            pltpu.VMEM((B, H), jnp.float32),
        ],
        compiler_params=pltpu.CompilerParams(
            dimension_semantics=("arbitrary",),
            vmem_limit_bytes=100 * 1024 * 1024,
        ),
    )(x_emb, w16, b, c0, h0)

    return cy_seq, hy_seq


# speedup vs baseline: 1.0097x; 1.0097x over previous
import jax
import jax.numpy as jnp
from jax.experimental import pallas as pl
from jax.experimental.pallas import tpu as pltpu

_TIME_CHUNK = 8


def _lstm_seq_kernel(x_ref,
                     w_ref,
                     b_ref,
                     c0_ref,
                     h0_ref,
                     cy_ref,
                     hy_ref,
                     c_st, h_st):
    Tc = x_ref.shape[0]
    H = c0_ref.shape[1]

    @pl.when(pl.program_id(0) == 0)
    def _():
        c_st[...] = c0_ref[...]
        h_st[...] = h0_ref[...]

    def step(t, carry):
        c, h = carry
        xh = jnp.concatenate([x_ref[t].astype(jnp.bfloat16),
                              h.astype(jnp.bfloat16)], axis=1)
        gates = jnp.dot(xh, w_ref[...],
                        preferred_element_type=jnp.float32) + b_ref[...]
        ingate     = 0.5 * jnp.tanh(0.5 * gates[:, 0 * H:1 * H]) + 0.5
        forgetgate = 0.5 * jnp.tanh(0.5 * gates[:, 1 * H:2 * H]) + 0.5
        cellgate   = jnp.tanh(gates[:, 2 * H:3 * H])
        outgate    = 0.5 * jnp.tanh(0.5 * gates[:, 3 * H:4 * H]) + 0.5
        cy = forgetgate * c + ingate * cellgate
        hy = outgate * jnp.tanh(cy)
        cy_ref[t] = cy
        hy_ref[t] = hy
        return (cy, hy)

    cy, hy = jax.lax.fori_loop(0, Tc, step, (c_st[...], h_st[...]),
                               unroll=8)
    c_st[...] = cy
    h_st[...] = hy


def kernel(tokens, c0, h0, table, wi, bi, wh, bh):
    T, B = tokens.shape
    V, H = table.shape
    Tc = _TIME_CHUNK if T % _TIME_CHUNK == 0 else T

    x_emb = jnp.take(table, tokens, axis=0)
    b = bi + bh
    w16 = jnp.concatenate([wi, wh], axis=0).astype(jnp.bfloat16)

    cy_seq, hy_seq = pl.pallas_call(
        _lstm_seq_kernel,
        out_shape=(jax.ShapeDtypeStruct((T, B, H), jnp.float32),
                   jax.ShapeDtypeStruct((T, B, H), jnp.float32)),
        grid=(T // Tc,),
        in_specs=[
            pl.BlockSpec((Tc, B, H),  lambda i: (i, 0, 0)),
            pl.BlockSpec((2 * H, 4 * H), lambda i: (0, 0)),
            pl.BlockSpec((1, 4 * H),  lambda i: (0, 0)),
            pl.BlockSpec((B, H),      lambda i: (0, 0)),
            pl.BlockSpec((B, H),      lambda i: (0, 0)),
        ],
        out_specs=(pl.BlockSpec((Tc, B, H), lambda i: (i, 0, 0)),
                   pl.BlockSpec((Tc, B, H), lambda i: (i, 0, 0))),
        scratch_shapes=[
            pltpu.VMEM((B, H), jnp.float32),
            pltpu.VMEM((B, H), jnp.float32),
        ],
        compiler_params=pltpu.CompilerParams(
            dimension_semantics=("arbitrary",),
            vmem_limit_bytes=100 * 1024 * 1024,
        ),
    )(x_emb, w16, b, c0, h0)

    return cy_seq, hy_seq


# speedup vs baseline: 1.1539x; 1.1428x over previous
import jax
import jax.numpy as jnp
from jax.experimental import pallas as pl
from jax.experimental.pallas import tpu as pltpu

_TIME_CHUNK = 8


def _lstm_seq_kernel(x_ref,
                     w_ref,
                     b_ref,
                     c0_ref,
                     h0_ref,
                     cy_ref,
                     hy_ref,
                     c_st, h_st):
    Tc = x_ref.shape[0]
    H = c0_ref.shape[1]

    @pl.when(pl.program_id(0) == 0)
    def _():
        c_st[...] = c0_ref[...]
        h_st[...] = h0_ref[...]

    def step(t, carry):
        c, h = carry
        xh = jnp.concatenate([x_ref[t].astype(jnp.bfloat16),
                              h.astype(jnp.bfloat16)], axis=1)
        gates = jnp.dot(xh, w_ref[...],
                        preferred_element_type=jnp.float32) + b_ref[...]
        ingate     = 0.5 * jnp.tanh(0.5 * gates[:, 0 * H:1 * H]) + 0.5
        forgetgate = 0.5 * jnp.tanh(0.5 * gates[:, 1 * H:2 * H]) + 0.5
        cellgate   = jnp.tanh(gates[:, 2 * H:3 * H])
        outgate    = 0.5 * jnp.tanh(0.5 * gates[:, 3 * H:4 * H]) + 0.5
        cy = forgetgate * c + ingate * cellgate
        hy = outgate * jnp.tanh(cy)
        cy_ref[t] = cy
        hy_ref[t] = hy
        return (cy, hy)

    cy, hy = jax.lax.fori_loop(0, Tc, step, (c_st[...], h_st[...]),
                               unroll=8)
    c_st[...] = cy
    h_st[...] = hy


def kernel(tokens, c0, h0, table, wi, bi, wh, bh):
    T, B = tokens.shape
    V, H = table.shape
    Tc = _TIME_CHUNK if T % _TIME_CHUNK == 0 else T

    x_emb = table.at[tokens].get(mode='promise_in_bounds')
    b = bi + bh
    w16 = jnp.concatenate([wi, wh], axis=0).astype(jnp.bfloat16)

    cy_seq, hy_seq = pl.pallas_call(
        _lstm_seq_kernel,
        out_shape=(jax.ShapeDtypeStruct((T, B, H), jnp.float32),
                   jax.ShapeDtypeStruct((T, B, H), jnp.float32)),
        grid=(T // Tc,),
        in_specs=[
            pl.BlockSpec((Tc, B, H),  lambda i: (i, 0, 0)),
            pl.BlockSpec((2 * H, 4 * H), lambda i: (0, 0)),
            pl.BlockSpec((1, 4 * H),  lambda i: (0, 0)),
            pl.BlockSpec((B, H),      lambda i: (0, 0)),
            pl.BlockSpec((B, H),      lambda i: (0, 0)),
        ],
        out_specs=(pl.BlockSpec((Tc, B, H), lambda i: (i, 0, 0)),
                   pl.BlockSpec((Tc, B, H), lambda i: (i, 0, 0))),
        scratch_shapes=[
            pltpu.VMEM((B, H), jnp.float32),
            pltpu.VMEM((B, H), jnp.float32),
        ],
        compiler_params=pltpu.CompilerParams(
            dimension_semantics=("arbitrary",),
            vmem_limit_bytes=100 * 1024 * 1024,
        ),
    )(x_emb, w16, b, c0, h0)

    return cy_seq, hy_seq
